# baseline (speedup 1.0000x reference)
import functools

import jax
import jax.numpy as jnp
from jax import lax
from jax.experimental import pallas as pl
from jax.experimental.pallas import tpu as pltpu
from jax.experimental.pallas import tpu_sc as plsc

_D = 64
_BATCH = 4096
_HIST = 50
_NW = 32
_C = 128
_NBUF = 5

_mesh = plsc.VectorSubcoreMesh(core_axis_name="c", subcore_axis_name="s")


@functools.partial(
    pl.kernel,
    mesh=_mesh,
    out_type=jax.ShapeDtypeStruct(
        (_HIST, _D // 8, _NW, 8, _C), jnp.float32
    ),
    scratch_types=[
        pltpu.VMEM((_HIST, _C), jnp.int32),
        pltpu.VMEM((_NBUF, _C, _D), jnp.float32),
        pltpu.VMEM((_NBUF, _D // 8, 8, _C + 1), jnp.float32),
        pltpu.SemaphoreType.DMA((_NBUF,)),
        pltpu.SemaphoreType.DMA((_NBUF,)),
    ],
    compiler_params=pltpu.CompilerParams(
        use_tc_tiling_on_sc=False,
        needs_layout_passes=False,
        disable_bounds_checks=True,
    ),
)
def _embed(table_hbm, x_hbm, out_hbm, idx_v, rows_v, blk_v, gsem, wsem):
    wid = lax.axis_index("s") * 2 + lax.axis_index("c")
    lanes = lax.broadcasted_iota(jnp.int32, (16,), 0)
    evecs = [lanes + (c * 16) for c in range(4)]
    trvecs = [lax.shift_right_logical(ev, 3) for ev in evecs]
    rvecs = [lax.bitwise_and(ev, jnp.full((16,), 7, jnp.int32)) for ev in evecs]

    pltpu.sync_copy(x_hbm.at[:, pl.ds(wid * _C, _C)], idx_v)

    def fire(h, b):
        pltpu.async_copy(table_hbm.at[idx_v.at[h]], rows_v.at[b], gsem.at[b])

    for b in range(_NBUF - 1):
        fire(b, b)

    def outer(t, carry):
        for b in range(_NBUF):
            h = _NBUF * t + b

            @pl.when(h + _NBUF - 1 < _HIST)
            def _():
                fire(h + _NBUF - 1, (b + _NBUF - 1) % _NBUF)

            pltpu.make_async_copy(
                table_hbm.at[idx_v.at[0]], rows_v.at[b], gsem.at[b]
            ).wait()

            @pl.when(h >= _NBUF)
            def _():
                pltpu.make_async_copy(
                    blk_v.at[b, :, :, pl.ds(0, _C)], out_hbm.at[0, :, 0],
                    wsem.at[b],
                ).wait()

            rows_ref = rows_v.at[b]
            blk_ref = blk_v.at[b]

            lv = jnp.zeros((16,), jnp.int32)
            one = jnp.ones((16,), jnp.int32)
            for g in range(0, _C, 4):
                loaded = []
                lvs = []
                for l in range(g, g + 4):
                    loaded.append(
                        tuple(rows_ref[l, pl.ds(c * 16, 16)] for c in range(4))
                    )
                    lvs.append(lv)
                    lv = lv + one
                for i in range(4):
                    for c in range(4):
                        plsc.store_scatter(
                            blk_ref, [trvecs[c], rvecs[c], lvs[i]], loaded[i][c]
                        )

            pltpu.async_copy(
                blk_v.at[b, :, :, pl.ds(0, _C)], out_hbm.at[h, :, wid],
                wsem.at[b],
            )
        return carry

    lax.fori_loop(0, _HIST // _NBUF, outer, 0)

    for b in range(_NBUF):
        pltpu.make_async_copy(
            blk_v.at[b, :, :, pl.ds(0, _C)], out_hbm.at[0, :, 0], wsem.at[b]
        ).wait()


def kernel(x, table):
    xw = x.T.astype(jnp.int32)
    out5 = _embed(table, xw)
    out = out5.transpose(2, 4, 0, 1, 3).reshape(_BATCH, _HIST, _D)
    return out


# speedup vs baseline: 1.0395x; 1.0395x over previous
import functools

import jax
import jax.numpy as jnp
from jax import lax
from jax.experimental import pallas as pl
from jax.experimental.pallas import tpu as pltpu
from jax.experimental.pallas import tpu_sc as plsc

_D = 64
_BATCH = 4096
_HIST = 50
_NW = 32
_C = 128
_NBUF = 5

_mesh = plsc.VectorSubcoreMesh(core_axis_name="c", subcore_axis_name="s")


@functools.partial(
    pl.kernel,
    mesh=_mesh,
    out_type=jax.ShapeDtypeStruct(
        (_HIST, _D // 8, _NW, 8, _C), jnp.float32
    ),
    scratch_types=[
        pltpu.VMEM((_HIST, _C), jnp.int32),
        pltpu.VMEM((_NBUF, _C, _D), jnp.float32),
        pltpu.VMEM((_NBUF, _D // 8, 8, _C + 1), jnp.float32),
        pltpu.SemaphoreType.DMA((_NBUF,)),
        pltpu.SemaphoreType.DMA((_NBUF,)),
    ],
    compiler_params=pltpu.CompilerParams(
        use_tc_tiling_on_sc=False,
        needs_layout_passes=False,
        disable_bounds_checks=True,
    ),
)
def _embed(table_hbm, x_hbm, out_hbm, idx_v, rows_v, blk_v, gsem, wsem):
    wid = lax.axis_index("s") * 2 + lax.axis_index("c")
    lanes = lax.broadcasted_iota(jnp.int32, (16,), 0)
    evecs = [lanes + (c * 16) for c in range(4)]
    trvecs = [lax.shift_right_logical(ev, 3) for ev in evecs]
    rvecs = [lax.bitwise_and(ev, jnp.full((16,), 7, jnp.int32)) for ev in evecs]

    pltpu.sync_copy(x_hbm.at[:, pl.ds(wid * _C, _C)], idx_v)

    def fire(h, b):
        pltpu.async_copy(table_hbm.at[idx_v.at[h]], rows_v.at[b], gsem.at[b])

    for b in range(_NBUF - 1):
        fire(b, b)

    def outer(t, carry):
        for b in range(_NBUF):
            h = _NBUF * t + b

            @pl.when(h + _NBUF - 1 < _HIST)
            def _():
                fire(h + _NBUF - 1, (b + _NBUF - 1) % _NBUF)

            pltpu.make_async_copy(
                table_hbm.at[idx_v.at[0]], rows_v.at[b], gsem.at[b]
            ).wait()

            @pl.when(h >= _NBUF)
            def _():
                pltpu.make_async_copy(
                    blk_v.at[b, :, :, pl.ds(0, _C)], out_hbm.at[0, :, 0],
                    wsem.at[b],
                ).wait()

            rows_ref = rows_v.at[b]
            blk_ref = blk_v.at[b]

            lv = jnp.zeros((16,), jnp.int32)
            one = jnp.ones((16,), jnp.int32)
            pend = None
            pend_lv = None
            for l in range(_C):
                cur = tuple(rows_ref[l, pl.ds(c * 16, 16)] for c in range(4))
                if pend is not None:
                    for c in range(4):
                        plsc.store_scatter(
                            blk_ref, [trvecs[c], rvecs[c], pend_lv], pend[c]
                        )
                pend, pend_lv = cur, lv
                lv = lv + one
            for c in range(4):
                plsc.store_scatter(
                    blk_ref, [trvecs[c], rvecs[c], pend_lv], pend[c]
                )

            pltpu.async_copy(
                blk_v.at[b, :, :, pl.ds(0, _C)], out_hbm.at[h, :, wid],
                wsem.at[b],
            )
        return carry

    lax.fori_loop(0, _HIST // _NBUF, outer, 0)

    for b in range(_NBUF):
        pltpu.make_async_copy(
            blk_v.at[b, :, :, pl.ds(0, _C)], out_hbm.at[0, :, 0], wsem.at[b]
        ).wait()


def kernel(x, table):
    xw = x.T.astype(jnp.int32)
    out5 = _embed(table, xw)
    out = out5.transpose(2, 4, 0, 1, 3).reshape(_BATCH, _HIST, _D)
    return out


# speedup vs baseline: 1.9197x; 1.8468x over previous
import functools

import jax
import jax.numpy as jnp
from jax import lax
from jax.experimental import pallas as pl
from jax.experimental.pallas import tpu as pltpu
from jax.experimental.pallas import tpu_sc as plsc

_D = 64
_BATCH = 4096
_HIST = 50
_NW = 32
_C = 128
_NBUF = 5

_mesh = plsc.VectorSubcoreMesh(core_axis_name="c", subcore_axis_name="s")


@functools.partial(
    pl.kernel,
    mesh=_mesh,
    out_type=jax.ShapeDtypeStruct(
        (_HIST, _D // 8, _NW, 8, _C), jnp.float32
    ),
    scratch_types=[
        pltpu.VMEM((_HIST, _C), jnp.int32),
        pltpu.VMEM((_NBUF, _C, _D), jnp.float32),
        pltpu.VMEM((_NBUF, _D // 8, 8, _C + 1), jnp.float32),
        pltpu.SemaphoreType.DMA((_NBUF,)),
        pltpu.SemaphoreType.DMA((_NBUF,)),
    ],
    compiler_params=pltpu.CompilerParams(
        use_tc_tiling_on_sc=False,
        needs_layout_passes=False,
        disable_bounds_checks=True,
    ),
)
def _embed(table_hbm, x_hbm, out_hbm, idx_v, rows_v, blk_v, gsem, wsem):
    wid = lax.axis_index("s") * 2 + lax.axis_index("c")
    lanes = lax.broadcasted_iota(jnp.int32, (16,), 0)
    evecs = [lanes + (c * 16) for c in range(4)]
    trvecs = [lax.shift_right_logical(ev, 3) for ev in evecs]
    rvecs = [lax.bitwise_and(ev, jnp.full((16,), 7, jnp.int32)) for ev in evecs]

    pltpu.sync_copy(x_hbm.at[:, pl.ds(wid * _C, _C)], idx_v)

    def fire(h, b):
        pltpu.async_copy(table_hbm.at[idx_v.at[h]], rows_v.at[b], gsem.at[b])

    for b in range(_NBUF - 1):
        fire(b, b)

    def outer(t, carry):
        for b in range(_NBUF):
            h = _NBUF * t + b

            @pl.when(h + _NBUF - 1 < _HIST)
            def _():
                fire(h + _NBUF - 1, (b + _NBUF - 1) % _NBUF)

            pltpu.make_async_copy(
                table_hbm.at[idx_v.at[0]], rows_v.at[b], gsem.at[b]
            ).wait()

            @pl.when(h >= _NBUF)
            def _():
                pltpu.make_async_copy(
                    blk_v.at[b, :, :, pl.ds(0, _C)], out_hbm.at[0, :, 0],
                    wsem.at[b],
                ).wait()

            rows_ref = rows_v.at[b]
            blk_ref = blk_v.at[b]

            one = jnp.ones((16,), jnp.int32)

            @plsc.parallel_loop(
                0, _C, step=1, unroll=8, carry=jnp.zeros((16,), jnp.int32)
            )
            def _transpose(l, lv):
                for c in range(4):
                    v = rows_ref[l, pl.ds(c * 16, 16)]
                    plsc.store_scatter(
                        blk_ref, [trvecs[c], rvecs[c], lv], v
                    )
                return lv + one

            pltpu.async_copy(
                blk_v.at[b, :, :, pl.ds(0, _C)], out_hbm.at[h, :, wid],
                wsem.at[b],
            )
        return carry

    lax.fori_loop(0, _HIST // _NBUF, outer, 0)

    for b in range(_NBUF):
        pltpu.make_async_copy(
            blk_v.at[b, :, :, pl.ds(0, _C)], out_hbm.at[0, :, 0], wsem.at[b]
        ).wait()


def kernel(x, table):
    xw = x.T.astype(jnp.int32)
    out5 = _embed(table, xw)
    out = out5.transpose(2, 4, 0, 1, 3).reshape(_BATCH, _HIST, _D)
    return out


# speedup vs baseline: 1.9220x; 1.0012x over previous
import functools

import jax
import jax.numpy as jnp
from jax import lax
from jax.experimental import pallas as pl
from jax.experimental.pallas import tpu as pltpu
from jax.experimental.pallas import tpu_sc as plsc

_D = 64
_BATCH = 4096
_HIST = 50
_NW = 32
_C = 128
_NBUF = 5

_mesh = plsc.VectorSubcoreMesh(core_axis_name="c", subcore_axis_name="s")


@functools.partial(
    pl.kernel,
    mesh=_mesh,
    out_type=jax.ShapeDtypeStruct(
        (_HIST, _D // 8, _NW, 8, _C), jnp.float32
    ),
    scratch_types=[
        pltpu.VMEM((_HIST, _C), jnp.int32),
        pltpu.VMEM((_NBUF, _C, _D), jnp.float32),
        pltpu.VMEM((_NBUF, _D // 8, 8, _C + 1), jnp.float32),
        pltpu.SemaphoreType.DMA((_NBUF,)),
        pltpu.SemaphoreType.DMA((_NBUF,)),
    ],
    compiler_params=pltpu.CompilerParams(
        use_tc_tiling_on_sc=False,
        needs_layout_passes=False,
        disable_bounds_checks=True,
    ),
)
def _embed(table_hbm, x_hbm, out_hbm, idx_v, rows_v, blk_v, gsem, wsem):
    wid = lax.axis_index("s") * 2 + lax.axis_index("c")
    lanes = lax.broadcasted_iota(jnp.int32, (16,), 0)
    evecs = [lanes + (c * 16) for c in range(4)]
    trvecs = [lax.shift_right_logical(ev, 3) for ev in evecs]
    rvecs = [lax.bitwise_and(ev, jnp.full((16,), 7, jnp.int32)) for ev in evecs]

    pltpu.sync_copy(x_hbm.at[:, pl.ds(wid * _C, _C)], idx_v)

    def fire(h, b):
        pltpu.async_copy(table_hbm.at[idx_v.at[h]], rows_v.at[b], gsem.at[b])

    for b in range(_NBUF - 1):
        fire(b, b)

    def outer(t, carry):
        for b in range(_NBUF):
            h = _NBUF * t + b

            @pl.when(h + _NBUF - 1 < _HIST)
            def _():
                fire(h + _NBUF - 1, (b + _NBUF - 1) % _NBUF)

            pltpu.make_async_copy(
                table_hbm.at[idx_v.at[0]], rows_v.at[b], gsem.at[b]
            ).wait()

            @pl.when(h >= _NBUF)
            def _():
                pltpu.make_async_copy(
                    blk_v.at[b, :, :, pl.ds(0, _C)], out_hbm.at[0, :, 0],
                    wsem.at[b],
                ).wait()

            rows_ref = rows_v.at[b]
            blk_ref = blk_v.at[b]

            one = jnp.ones((16,), jnp.int32)

            @plsc.parallel_loop(
                0, _C, step=1, unroll=8, carry=jnp.zeros((16,), jnp.int32)
            )
            def _transpose(l, lv):
                for c in range(4):
                    v = rows_ref[l, pl.ds(c * 16, 16)]
                    plsc.store_scatter(
                        blk_ref, [trvecs[c], rvecs[c], lv], v
                    )
                return lv + one

            pltpu.async_copy(
                blk_v.at[b, :, :, pl.ds(0, _C)], out_hbm.at[h, :, wid],
                wsem.at[b],
            )
        return carry

    lax.fori_loop(0, _HIST // _NBUF, outer, 0)

    for b in range(_NBUF):
        pltpu.make_async_copy(
            blk_v.at[b, :, :, pl.ds(0, _C)], out_hbm.at[0, :, 0], wsem.at[b]
        ).wait()


def kernel(x, table):
    xw = lax.optimization_barrier(x.T.astype(jnp.int32))
    out5 = _embed(table, xw)
    out = out5.transpose(2, 4, 0, 1, 3).reshape(_BATCH, _HIST, _D)
    return out
